# baseline (speedup 1.0000x reference)
import functools

import jax
import jax.numpy as jnp
from jax import lax
from jax.experimental import pallas as pl
from jax.experimental.pallas import tpu as pltpu
from jax.experimental.pallas import tpu_sc as plsc

_B = 16384
_D = 64
_NC = 2
_NS = 16
_NW = _NC * _NS
_BPW = _B // _NW
_CHUNK = 128
_NCH = _BPW // _CHUNK


def _tec_body(emb_e, emb_rel, s_idx, r_idx, o_idx, out,
              s_iv, r_iv, o_iv, s_buf, r_buf, o_buf, sem):
  wid = lax.axis_index("s") * _NC + lax.axis_index("c")

  pltpu.sync_copy(s_idx.at[wid], s_iv)
  pltpu.sync_copy(r_idx.at[wid], r_iv)
  pltpu.sync_copy(o_idx.at[wid], o_iv)

  copies = []
  for ci in range(_NCH):
    rows = pl.ds(ci * _CHUNK, _CHUNK)
    copies.append(pltpu.async_copy(emb_e.at[s_iv.at[ci]], s_buf.at[rows], sem))
    copies.append(pltpu.async_copy(emb_rel.at[r_iv.at[ci]], r_buf.at[rows], sem))
    copies.append(pltpu.async_copy(emb_e.at[o_iv.at[ci]], o_buf.at[rows], sem))
  for c in copies:
    c.wait()

  def body(i, carry):
    for j in range(_D // 16):
      sl = pl.ds(j * 16, 16)
      s_buf[i, sl] = s_buf[i, sl] * r_buf[i, sl] * o_buf[i, sl]
    return carry

  lax.fori_loop(0, _BPW, body, 0)

  pltpu.sync_copy(s_buf, out.at[pl.ds(wid * _BPW, _BPW)])


@jax.jit
def _dist_mult(emb_e, emb_rel, s_idx, r_idx, o_idx):
  mesh = plsc.VectorSubcoreMesh(core_axis_name="c", subcore_axis_name="s")
  kern = functools.partial(
      pl.kernel,
      mesh=mesh,
      out_type=jax.ShapeDtypeStruct((_B, _D), jnp.float32),
      compiler_params=pltpu.CompilerParams(use_tc_tiling_on_sc=False),
      scratch_types=[
          pltpu.VMEM((_NCH, _CHUNK), jnp.int32),
          pltpu.VMEM((_NCH, _CHUNK), jnp.int32),
          pltpu.VMEM((_NCH, _CHUNK), jnp.int32),
          pltpu.VMEM((_BPW, _D), jnp.float32),
          pltpu.VMEM((_BPW, _D), jnp.float32),
          pltpu.VMEM((_BPW, _D), jnp.float32),
          pltpu.SemaphoreType.DMA,
      ],
  )(_tec_body)
  return kern(emb_e, emb_rel, s_idx, r_idx, o_idx)


def kernel(emb_e, emb_rel, triplets):
  s_idx = triplets[:, 0].reshape(_NW, _NCH, _CHUNK)
  r_idx = triplets[:, 1].reshape(_NW, _NCH, _CHUNK)
  o_idx = triplets[:, 2].reshape(_NW, _NCH, _CHUNK)
  return _dist_mult(emb_e, emb_rel, s_idx, r_idx, o_idx)


# speedup vs baseline: 1.1679x; 1.1679x over previous
import functools

import jax
import jax.numpy as jnp
from jax import lax
from jax.experimental import pallas as pl
from jax.experimental.pallas import tpu as pltpu
from jax.experimental.pallas import tpu_sc as plsc

_B = 16384
_D = 64
_NW = 32
_BPW = _B // _NW
_CH = 16
_NSUB = 8
_NBIG = _BPW // (_CH * _NSUB)


def _tec_body(tab, rel, s_i, r_i, o_i, out_t,
              rbuf, sbuf, obuf, out_v, s_vv, r_vv, o_vv, sem, relsem):
  wid = lax.axis_index("s") * 2 + lax.axis_index("c")
  base = wid * _BPW
  iota = lax.iota(jnp.int32, 16)

  def big(bi, carry):
    def sub(m, carry2):
      off = base + bi * (_CH * _NSUB) + m * _CH
      pltpu.sync_copy(s_i.at[pl.ds(off, _CH)], s_vv)
      pltpu.sync_copy(r_i.at[pl.ds(off, _CH)], r_vv)
      pltpu.sync_copy(o_i.at[pl.ds(off, _CH)], o_vv)
      s_vec = s_vv[...]
      o_vec = o_vv[...]
      r_vec = r_vv[...]
      descs = []
      for j in range(_CH):
        sg = pl.multiple_of((s_vec[j] >> 3) * 8, 8)
        og = pl.multiple_of((o_vec[j] >> 3) * 8, 8)
        rg = pl.multiple_of((r_vec[j] >> 3) * 8, 8)
        dst = pl.ds(j * 8, 8)
        descs.append(pltpu.async_copy(
            tab.at[pl.ds(sg, 8), :], sbuf.at[dst, :], sem))
        descs.append(pltpu.async_copy(
            tab.at[pl.ds(og, 8), :], obuf.at[dst, :], sem))
        descs.append(pltpu.async_copy(
            rel.at[pl.ds(rg, 8), :], rbuf.at[dst, :], sem))
      for dsc in descs:
        dsc.wait()
      rows_s = iota * 8 + (s_vec & 7)
      rows_o = iota * 8 + (o_vec & 7)
      rows_r = iota * 8 + (r_vec & 7)
      for d in range(_D):
        dv = jnp.full((16,), d, jnp.int32)
        sv = plsc.load_gather(sbuf, [rows_s, dv])
        ov = plsc.load_gather(obuf, [rows_o, dv])
        rv = plsc.load_gather(rbuf, [rows_r, dv])
        out_v[d, pl.ds(m * _CH, _CH)] = sv * rv * ov
      return carry2

    lax.fori_loop(0, _NSUB, sub, 0)
    tile_off = base + bi * (_CH * _NSUB)
    pltpu.sync_copy(out_v, out_t.at[:, pl.ds(pl.multiple_of(tile_off, 128),
                                             _CH * _NSUB)])
    return carry

  lax.fori_loop(0, _NBIG, big, 0)


@jax.jit
def _dist_mult(tab, rel, s_i, r_i, o_i):
  mesh = plsc.VectorSubcoreMesh(core_axis_name="c", subcore_axis_name="s")
  kern = functools.partial(
      pl.kernel,
      mesh=mesh,
      out_type=jax.ShapeDtypeStruct((_D, _B), jnp.float32),
      compiler_params=pltpu.CompilerParams(needs_layout_passes=False),
      scratch_types=[
          pltpu.VMEM((_CH * 8, _D), jnp.float32),
          pltpu.VMEM((_CH * 8, _D), jnp.float32),
          pltpu.VMEM((_CH * 8, _D), jnp.float32),
          pltpu.VMEM((_D, _CH * _NSUB), jnp.float32),
          pltpu.VMEM((_CH,), jnp.int32),
          pltpu.VMEM((_CH,), jnp.int32),
          pltpu.VMEM((_CH,), jnp.int32),
          pltpu.SemaphoreType.DMA,
          pltpu.SemaphoreType.DMA,
      ],
  )(_tec_body)
  return kern(tab, rel, s_i, r_i, o_i)


def kernel(emb_e, emb_rel, triplets):
  out_t = _dist_mult(emb_e, emb_rel,
                     triplets[:, 0], triplets[:, 1], triplets[:, 2])
  return out_t.T


# speedup vs baseline: 1.3510x; 1.1568x over previous
import functools

import jax
import jax.numpy as jnp
from jax import lax
from jax.experimental import pallas as pl
from jax.experimental.pallas import tpu as pltpu
from jax.experimental.pallas import tpu_sc as plsc

_B = 16384
_D = 64
_NW = 32
_BPW = _B // _NW
_CH = 16
_NSUB = _BPW // _CH


def _tec_body(tab, rel, s_i, r_i, o_i, out_t,
              sbuf, rbuf, obuf, out_v, s_vv, r_vv, o_vv, sem):
  wid = lax.axis_index("s") * 2 + lax.axis_index("c")
  base = wid * _BPW
  iota = lax.iota(jnp.int32, 16)

  def issue(m):
    par = m & 1
    off = base + m * _CH
    pltpu.sync_copy(s_i.at[pl.ds(off, _CH)], s_vv.at[par])
    pltpu.sync_copy(r_i.at[pl.ds(off, _CH)], r_vv.at[par])
    pltpu.sync_copy(o_i.at[pl.ds(off, _CH)], o_vv.at[par])
    s_vec = s_vv[par, :]
    r_vec = r_vv[par, :]
    o_vec = o_vv[par, :]
    for j in range(_CH):
      sg = pl.multiple_of((s_vec[j] >> 3) * 8, 8)
      og = pl.multiple_of((o_vec[j] >> 3) * 8, 8)
      rg = pl.multiple_of((r_vec[j] >> 3) * 8, 8)
      dst = pl.ds(j * 8, 8)
      pltpu.async_copy(tab.at[pl.ds(sg, 8), :], sbuf.at[par, dst, :], sem.at[par])
      pltpu.async_copy(tab.at[pl.ds(og, 8), :], obuf.at[par, dst, :], sem.at[par])
      pltpu.async_copy(rel.at[pl.ds(rg, 8), :], rbuf.at[par, dst, :], sem.at[par])

  def assemble(m):
    par = m & 1
    pltpu.make_async_copy(tab.at[pl.ds(0, 128), :], sbuf.at[par], sem.at[par]).wait()
    pltpu.make_async_copy(tab.at[pl.ds(0, 128), :], obuf.at[par], sem.at[par]).wait()
    pltpu.make_async_copy(tab.at[pl.ds(0, 128), :], rbuf.at[par], sem.at[par]).wait()
    s_vec = s_vv[par, :]
    r_vec = r_vv[par, :]
    o_vec = o_vv[par, :]
    rows_s = iota * 8 + (s_vec & 7)
    rows_o = iota * 8 + (o_vec & 7)
    rows_r = iota * 8 + (r_vec & 7)
    pv = jnp.full((16,), par, jnp.int32)
    col = (m % 8) * _CH
    for d in range(_D):
      dv = jnp.full((16,), d, jnp.int32)
      sv = plsc.load_gather(sbuf, [pv, rows_s, dv])
      ov = plsc.load_gather(obuf, [pv, rows_o, dv])
      rv = plsc.load_gather(rbuf, [pv, rows_r, dv])
      out_v[d, pl.ds(col, _CH)] = sv * rv * ov
    @pl.when(m % 8 == 7)
    def _():
      tile_off = pl.multiple_of(base + (m - 7) * _CH, 128)
      pltpu.sync_copy(out_v, out_t.at[:, pl.ds(tile_off, 128)])

  def step(m, carry):
    @pl.when(m < _NSUB)
    def _():
      issue(m)

    @pl.when(m > 0)
    def _():
      assemble(m - 1)

    return carry

  lax.fori_loop(0, _NSUB + 1, step, 0)


@jax.jit
def _dist_mult(tab, rel, s_i, r_i, o_i):
  mesh = plsc.VectorSubcoreMesh(core_axis_name="c", subcore_axis_name="s")
  kern = functools.partial(
      pl.kernel,
      mesh=mesh,
      out_type=jax.ShapeDtypeStruct((_D, _B), jnp.float32),
      compiler_params=pltpu.CompilerParams(needs_layout_passes=False),
      scratch_types=[
          pltpu.VMEM((2, _CH * 8, _D), jnp.float32),
          pltpu.VMEM((2, _CH * 8, _D), jnp.float32),
          pltpu.VMEM((2, _CH * 8, _D), jnp.float32),
          pltpu.VMEM((_D, 128), jnp.float32),
          pltpu.VMEM((2, _CH), jnp.int32),
          pltpu.VMEM((2, _CH), jnp.int32),
          pltpu.VMEM((2, _CH), jnp.int32),
          pltpu.SemaphoreType.DMA((2,)),
      ],
  )(_tec_body)
  return kern(tab, rel, s_i, r_i, o_i)


def kernel(emb_e, emb_rel, triplets):
  out_t = _dist_mult(emb_e, emb_rel,
                     triplets[:, 0], triplets[:, 1], triplets[:, 2])
  return out_t.T


# speedup vs baseline: 1.8339x; 1.3574x over previous
import functools

import jax
import jax.numpy as jnp
from jax import lax
from jax.experimental import pallas as pl
from jax.experimental.pallas import tpu as pltpu
from jax.experimental.pallas import tpu_sc as plsc

_B = 16384
_D = 64
_NW = 32
_BPW = _B // _NW
_CH = 16
_NSUB = _BPW // _CH


def _tec_body(tab, rel, s_i, r_i, o_i, out_t,
              sbuf, rbuf, obuf, out_v, s_vv, r_vv, o_vv, sem):
  wid = lax.axis_index("s") * 2 + lax.axis_index("c")
  base = wid * _BPW
  iota = lax.iota(jnp.int32, 16)

  def issue(m):
    par = m & 1
    off = base + m * _CH
    pltpu.sync_copy(s_i.at[pl.ds(off, _CH)], s_vv.at[par])
    pltpu.sync_copy(r_i.at[pl.ds(off, _CH)], r_vv.at[par])
    pltpu.sync_copy(o_i.at[pl.ds(off, _CH)], o_vv.at[par])
    s_vec = s_vv[par, :]
    r_vec = r_vv[par, :]
    o_vec = o_vv[par, :]
    for j in range(_CH):
      sg = s_vec[j] >> 3
      og = o_vec[j] >> 3
      rg = r_vec[j] >> 3
      dst = pl.ds(j * 8, 8)
      pltpu.async_copy(tab.at[sg], sbuf.at[par, dst, :], sem.at[par])
      pltpu.async_copy(tab.at[og], obuf.at[par, dst, :], sem.at[par])
      pltpu.async_copy(rel.at[rg], rbuf.at[par, dst, :], sem.at[par])

  def assemble(m):
    par = m & 1
    pltpu.make_async_copy(tab.at[pl.ds(0, 16), :, :], sbuf.at[par].reshape(16, 8, _D), sem.at[par]).wait()
    pltpu.make_async_copy(tab.at[pl.ds(0, 16), :, :], obuf.at[par].reshape(16, 8, _D), sem.at[par]).wait()
    pltpu.make_async_copy(tab.at[pl.ds(0, 16), :, :], rbuf.at[par].reshape(16, 8, _D), sem.at[par]).wait()
    s_vec = s_vv[par, :]
    r_vec = r_vv[par, :]
    o_vec = o_vv[par, :]
    rows_s = iota * 8 + (s_vec & 7)
    rows_o = iota * 8 + (o_vec & 7)
    rows_r = iota * 8 + (r_vec & 7)
    pv = jnp.full((16,), par, jnp.int32)
    col = (m % 8) * _CH
    for d in range(_D):
      dv = jnp.full((16,), d, jnp.int32)
      sv = plsc.load_gather(sbuf, [pv, rows_s, dv])
      ov = plsc.load_gather(obuf, [pv, rows_o, dv])
      rv = plsc.load_gather(rbuf, [pv, rows_r, dv])
      out_v[d, pl.ds(col, _CH)] = sv * rv * ov
    @pl.when(m % 8 == 7)
    def _():
      tile_off = pl.multiple_of(base + (m - 7) * _CH, 128)
      pltpu.sync_copy(out_v, out_t.at[:, pl.ds(tile_off, 128)])

  def step(m, carry):
    @pl.when(m < _NSUB)
    def _():
      issue(m)

    @pl.when(m > 0)
    def _():
      assemble(m - 1)

    return carry

  lax.fori_loop(0, _NSUB + 1, step, 0)


@jax.jit
def _dist_mult(tab, rel, s_i, r_i, o_i):
  mesh = plsc.VectorSubcoreMesh(core_axis_name="c", subcore_axis_name="s")
  kern = functools.partial(
      pl.kernel,
      mesh=mesh,
      out_type=jax.ShapeDtypeStruct((_D, _B), jnp.float32),
      compiler_params=pltpu.CompilerParams(needs_layout_passes=False),
      scratch_types=[
          pltpu.VMEM((2, _CH * 8, _D), jnp.float32),
          pltpu.VMEM((2, _CH * 8, _D), jnp.float32),
          pltpu.VMEM((2, _CH * 8, _D), jnp.float32),
          pltpu.VMEM((_D, 128), jnp.float32),
          pltpu.VMEM((2, _CH), jnp.int32),
          pltpu.VMEM((2, _CH), jnp.int32),
          pltpu.VMEM((2, _CH), jnp.int32),
          pltpu.SemaphoreType.DMA((2,)),
      ],
  )(_tec_body)
  return kern(tab, rel, s_i, r_i, o_i)


def kernel(emb_e, emb_rel, triplets):
  tab3 = emb_e.reshape(125000, 8, _D)
  rel3 = emb_rel.reshape(125, 8, _D)
  out_t = _dist_mult(tab3, rel3,
                     triplets[:, 0], triplets[:, 1], triplets[:, 2])
  return out_t.T


# speedup vs baseline: 1.9894x; 1.0848x over previous
import functools

import jax
import jax.numpy as jnp
from jax import lax
from jax.experimental import pallas as pl
from jax.experimental.pallas import tpu as pltpu
from jax.experimental.pallas import tpu_sc as plsc

_B = 16384
_D = 64
_NW = 32
_BPW = _B // _NW
_CH = 16
_NSUB = _BPW // _CH
_NBUF = 2


def _tec_body(tab, rel, s_i, r_i, o_i, out_t,
              sbuf, rbuf, obuf, out_v, s_all, r_all, o_all, sem, osem):
  wid = lax.axis_index("s") * 2 + lax.axis_index("c")
  base = wid * _BPW
  iota = lax.iota(jnp.int32, 16)

  pltpu.sync_copy(s_i.at[pl.ds(base, _BPW)], s_all)
  pltpu.sync_copy(r_i.at[pl.ds(base, _BPW)], r_all)
  pltpu.sync_copy(o_i.at[pl.ds(base, _BPW)], o_all)

  def issue(m):
    par = lax.rem(m, _NBUF)
    sl = pl.ds(m * _CH, _CH)
    s_vec = s_all[sl]
    r_vec = r_all[sl]
    o_vec = o_all[sl]
    for j in range(_CH):
      sg = s_vec[j] >> 3
      og = o_vec[j] >> 3
      rg = r_vec[j] >> 3
      dst = pl.ds(j * 8, 8)
      pltpu.async_copy(tab.at[sg], sbuf.at[par, dst, :], sem.at[par])
      pltpu.async_copy(tab.at[og], obuf.at[par, dst, :], sem.at[par])
      pltpu.async_copy(rel.at[rg], rbuf.at[par, dst, :], sem.at[par])

  def assemble(m):
    par = lax.rem(m, _NBUF)
    pltpu.make_async_copy(tab.at[pl.ds(0, 16), :, :],
                          sbuf.at[par].reshape(16, 8, _D), sem.at[par]).wait()
    pltpu.make_async_copy(tab.at[pl.ds(0, 16), :, :],
                          obuf.at[par].reshape(16, 8, _D), sem.at[par]).wait()
    pltpu.make_async_copy(tab.at[pl.ds(0, 16), :, :],
                          rbuf.at[par].reshape(16, 8, _D), sem.at[par]).wait()
    sl = pl.ds(m * _CH, _CH)
    s_vec = s_all[sl]
    r_vec = r_all[sl]
    o_vec = o_all[sl]
    rows_s = iota * 8 + (s_vec & 7)
    rows_o = iota * 8 + (o_vec & 7)
    rows_r = iota * 8 + (r_vec & 7)
    tpar = lax.rem(m // 8, 2)
    pv = jnp.full((16,), par, jnp.int32)
    tv = jnp.full((16,), tpar, jnp.int32)
    col = (m % 8) * _CH
    for d in range(_D):
      dv = jnp.full((16,), d, jnp.int32)
      sv = plsc.load_gather(sbuf, [pv, rows_s, dv])
      ov = plsc.load_gather(obuf, [pv, rows_o, dv])
      rv = plsc.load_gather(rbuf, [pv, rows_r, dv])
      out_v[tpar, d, pl.ds(col, _CH)] = sv * rv * ov

    @pl.when(m % 8 == 7)
    def _():
      t = (m - 7) // 8

      @pl.when(t > 0)
      def _():
        pltpu.make_async_copy(out_v.at[0], out_t.at[:, pl.ds(0, 128)],
                              osem).wait()

      tile_off = pl.multiple_of(base + (m - 7) * _CH, 128)
      pltpu.async_copy(out_v.at[tpar], out_t.at[:, pl.ds(tile_off, 128)],
                       osem)

  def step(m, carry):
    @pl.when(m < _NSUB)
    def _():
      issue(m)

    @pl.when(m >= (_NBUF - 1))
    def _():
      assemble(m - (_NBUF - 1))

    return carry

  lax.fori_loop(0, _NSUB + _NBUF - 1, step, 0)
  pltpu.make_async_copy(out_v.at[0], out_t.at[:, pl.ds(0, 128)], osem).wait()


@jax.jit
def _dist_mult(tab, rel, s_i, r_i, o_i):
  mesh = plsc.VectorSubcoreMesh(core_axis_name="c", subcore_axis_name="s")
  kern = functools.partial(
      pl.kernel,
      mesh=mesh,
      out_type=jax.ShapeDtypeStruct((_D, _B), jnp.float32),
      compiler_params=pltpu.CompilerParams(needs_layout_passes=False),
      scratch_types=[
          pltpu.VMEM((_NBUF, _CH * 8, _D), jnp.float32),
          pltpu.VMEM((_NBUF, _CH * 8, _D), jnp.float32),
          pltpu.VMEM((_NBUF, _CH * 8, _D), jnp.float32),
          pltpu.VMEM((2, _D, 128), jnp.float32),
          pltpu.VMEM((_BPW,), jnp.int32),
          pltpu.VMEM((_BPW,), jnp.int32),
          pltpu.VMEM((_BPW,), jnp.int32),
          pltpu.SemaphoreType.DMA((_NBUF,)),
          pltpu.SemaphoreType.DMA,
      ],
  )(_tec_body)
  return kern(tab, rel, s_i, r_i, o_i)


def kernel(emb_e, emb_rel, triplets):
  tab3 = emb_e.reshape(125000, 8, _D)
  rel3 = emb_rel.reshape(125, 8, _D)
  out_t = _dist_mult(tab3, rel3,
                     triplets[:, 0], triplets[:, 1], triplets[:, 2])
  return out_t.T


# speedup vs baseline: 1.9902x; 1.0004x over previous
import functools

import jax
import jax.numpy as jnp
from jax import lax
from jax.experimental import pallas as pl
from jax.experimental.pallas import tpu as pltpu
from jax.experimental.pallas import tpu_sc as plsc

_B = 16384
_D = 64
_NW = 32
_BPW = _B // _NW
_CH = 16
_NSUB = _BPW // _CH
_NBUF = 2


def _tec_body(tab, rel, s_i, r_i, o_i, out_t,
              sbuf, rbuf, obuf, out_v, s_all, r_all, o_all, sem, osem):
  wid = lax.axis_index("s") * 2 + lax.axis_index("c")
  base = wid * _BPW
  iota = lax.iota(jnp.int32, 16)

  pltpu.sync_copy(s_i.at[pl.ds(base, _BPW)], s_all)
  pltpu.sync_copy(r_i.at[pl.ds(base, _BPW)], r_all)
  pltpu.sync_copy(o_i.at[pl.ds(base, _BPW)], o_all)

  def issue(m):
    par = lax.rem(m, _NBUF)
    pb = par * (_CH * 8)
    sl = pl.ds(m * _CH, _CH)
    s_vec = s_all[sl]
    r_vec = r_all[sl]
    o_vec = o_all[sl]
    for j in range(_CH):
      sg = s_vec[j] >> 3
      og = o_vec[j] >> 3
      rg = r_vec[j] >> 3
      dst = pl.ds(pb + j * 8, 8)
      pltpu.async_copy(tab.at[sg], sbuf.at[dst, :], sem.at[par])
      pltpu.async_copy(tab.at[og], obuf.at[dst, :], sem.at[par])
      pltpu.async_copy(rel.at[rg], rbuf.at[dst, :], sem.at[par])

  def assemble(m):
    par = lax.rem(m, _NBUF)
    pb = par * (_CH * 8)
    drain = pl.ds(pl.multiple_of(pb, 8), _CH * 8)
    pltpu.make_async_copy(tab.at[pl.ds(0, 16), :, :],
                          sbuf.at[drain, :].reshape(16, 8, _D),
                          sem.at[par]).wait()
    pltpu.make_async_copy(tab.at[pl.ds(0, 16), :, :],
                          obuf.at[drain, :].reshape(16, 8, _D),
                          sem.at[par]).wait()
    pltpu.make_async_copy(tab.at[pl.ds(0, 16), :, :],
                          rbuf.at[drain, :].reshape(16, 8, _D),
                          sem.at[par]).wait()
    sl = pl.ds(m * _CH, _CH)
    s_vec = s_all[sl]
    r_vec = r_all[sl]
    o_vec = o_all[sl]
    pbv = jnp.full((16,), par * (_CH * 8), jnp.int32) + iota * 8
    flat_s = pbv + (s_vec & 7)
    flat_o = pbv + (o_vec & 7)
    flat_r = pbv + (r_vec & 7)
    tpar = lax.rem(m // 8, 2)
    pv = jnp.full((16,), par, jnp.int32)
    tv = jnp.full((16,), tpar, jnp.int32)
    col = (m % 8) * _CH
    for d in range(_D):
      dv = jnp.full((16,), d, jnp.int32)
      sv = plsc.load_gather(sbuf, [flat_s, dv])
      ov = plsc.load_gather(obuf, [flat_o, dv])
      rv = plsc.load_gather(rbuf, [flat_r, dv])
      out_v[tpar, d, pl.ds(col, _CH)] = sv * rv * ov

    @pl.when(m % 8 == 7)
    def _():
      t = (m - 7) // 8

      @pl.when(t > 0)
      def _():
        pltpu.make_async_copy(out_v.at[0], out_t.at[:, pl.ds(0, 128)],
                              osem).wait()

      tile_off = pl.multiple_of(base + (m - 7) * _CH, 128)
      pltpu.async_copy(out_v.at[tpar], out_t.at[:, pl.ds(tile_off, 128)],
                       osem)

  def step(m, carry):
    @pl.when(m < _NSUB)
    def _():
      issue(m)

    @pl.when(m >= (_NBUF - 1))
    def _():
      assemble(m - (_NBUF - 1))

    return carry

  lax.fori_loop(0, _NSUB + _NBUF - 1, step, 0)
  pltpu.make_async_copy(out_v.at[0], out_t.at[:, pl.ds(0, 128)], osem).wait()


@jax.jit
def _dist_mult(tab, rel, s_i, r_i, o_i):
  mesh = plsc.VectorSubcoreMesh(core_axis_name="c", subcore_axis_name="s")
  kern = functools.partial(
      pl.kernel,
      mesh=mesh,
      out_type=jax.ShapeDtypeStruct((_D, _B), jnp.float32),
      compiler_params=pltpu.CompilerParams(needs_layout_passes=False),
      scratch_types=[
          pltpu.VMEM((_NBUF * _CH * 8, _D), jnp.float32),
          pltpu.VMEM((_NBUF * _CH * 8, _D), jnp.float32),
          pltpu.VMEM((_NBUF * _CH * 8, _D), jnp.float32),
          pltpu.VMEM((2, _D, 128), jnp.float32),
          pltpu.VMEM((_BPW,), jnp.int32),
          pltpu.VMEM((_BPW,), jnp.int32),
          pltpu.VMEM((_BPW,), jnp.int32),
          pltpu.SemaphoreType.DMA((_NBUF,)),
          pltpu.SemaphoreType.DMA,
      ],
  )(_tec_body)
  return kern(tab, rel, s_i, r_i, o_i)


def kernel(emb_e, emb_rel, triplets):
  tab3 = emb_e.reshape(125000, 8, _D)
  rel3 = emb_rel.reshape(125, 8, _D)
  out_t = _dist_mult(tab3, rel3,
                     triplets[:, 0], triplets[:, 1], triplets[:, 2])
  return out_t.T
